# baseline (speedup 1.0000x reference)
import jax
import jax.numpy as jnp
from jax.experimental import pallas as pl
from jax.experimental.pallas import tpu as pltpu

_SLOPE = 0.3
_EPS = 1e-5


def _leaky(v):
    return jnp.where(v >= 0, v, _SLOPE * v)


def _block_kernel(x_ref, w1_ref, t1_ref, w2_ref, t2_ref, o_ref, xw_ref, yw_ref):
    NB, D, H, W, C = x_ref.shape
    NQ = W // 2
    KW = 4 * C
    NC = 2 * C
    M = D * H * NQ
    bf16 = jnp.bfloat16

    zd = jnp.zeros((1, H + 2, NQ, KW), bf16)
    zh = jnp.zeros((D, 1, NQ, KW), bf16)
    zc = jnp.zeros((D, H, 1, C), bf16)
    for ref in (xw_ref, yw_ref):
        ref[0:1, :, :, :] = zd
        ref[D + 1:D + 2, :, :, :] = zd
        ref[1:1 + D, 0:1, :, :] = zh
        ref[1:1 + D, H + 1:H + 2, :, :] = zh
        ref[1:1 + D, 1:1 + H, 0:1, 0:C] = zc
        ref[1:1 + D, 1:1 + H, NQ - 1:NQ, KW - C:KW] = zc

    def conv(src_ref, w_ref):
        acc = jnp.zeros((M, NC), jnp.float32)
        for t in range(9):
            kd, kh = t // 3, t % 3
            lhs = src_ref[kd:kd + D, kh:kh + H, :, :].reshape(M, KW)
            acc = acc + jnp.dot(lhs, w_ref[t],
                                preferred_element_type=jnp.float32)
        return acc

    def body(i, carry):
        xb = x_ref[i].astype(bf16)
        for q in range(1, NQ - 1):
            xw_ref[1:1 + D, 1:1 + H, q:q + 1, :] = (
                xb[:, :, 2 * q - 1:2 * q + 3, :].reshape(D, H, 1, KW))
        xw_ref[1:1 + D, 1:1 + H, 0:1, C:KW] = (
            xb[:, :, 0:3, :].reshape(D, H, 1, 3 * C))
        xw_ref[1:1 + D, 1:1 + H, NQ - 1:NQ, 0:3 * C] = (
            xb[:, :, W - 3:W, :].reshape(D, H, 1, 3 * C))

        y = _leaky(conv(xw_ref, w1_ref) + t1_ref[...])
        yb = y.astype(bf16).reshape(D, H, NQ, 2, C)
        yw_ref[1:1 + D, 1:1 + H, :, C:3 * C] = yb.reshape(D, H, NQ, NC)
        yw_ref[1:1 + D, 1:1 + H, 0:NQ - 1, 3 * C:KW] = yb[:, :, 1:NQ, 0, :]
        yw_ref[1:1 + D, 1:1 + H, 1:NQ, 0:C] = yb[:, :, 0:NQ - 1, 1, :]

        z = conv(yw_ref, w2_ref) + t2_ref[...] + x_ref[i].reshape(M, NC)
        o_ref[i] = _leaky(z).reshape(D, H, W, C)
        return carry

    jax.lax.fori_loop(0, NB, body, 0)


def _build_call(N, D, H, W, C, NB):
    NQ = W // 2
    KW, NC = 4 * C, 2 * C
    vol = pl.BlockSpec((NB, D, H, W, C), lambda n: (n, 0, 0, 0, 0))
    wspec = pl.BlockSpec((9, KW, NC), lambda n: (0, 0, 0))
    tspec = pl.BlockSpec((1, NC), lambda n: (0, 0))
    return pl.pallas_call(
        _block_kernel,
        out_shape=jax.ShapeDtypeStruct((N, D, H, W, C), jnp.float32),
        grid=(N // NB,),
        in_specs=[vol, wspec, tspec, wspec, tspec],
        out_specs=vol,
        scratch_shapes=[
            pltpu.VMEM((D + 2, H + 2, NQ, KW), jnp.bfloat16),
            pltpu.VMEM((D + 2, H + 2, NQ, KW), jnp.bfloat16),
        ],
        compiler_params=pltpu.CompilerParams(
            dimension_semantics=("parallel",),
            vmem_limit_bytes=52 * 1024 * 1024,
        ),
    )


def _fold_band(w, conv_b, gamma, beta, mean, var, C):
    s = gamma * jax.lax.rsqrt(var + _EPS)
    t = conv_b * s + beta - mean * s
    wt = jnp.transpose(w * s[:, None, None, None, None],
                       (2, 3, 4, 1, 0))
    band = jnp.zeros((3, 3, 4, C, 2, C), jnp.float32)
    for wq in range(2):
        for kw in range(3):
            band = band.at[:, :, wq + kw, :, wq, :].set(wt[:, :, kw])
    band = band.reshape(9, 4 * C, 2 * C).astype(jnp.bfloat16)
    tcol = jnp.concatenate([t, t]).reshape(1, 2 * C).astype(jnp.float32)
    return band, tcol


def kernel(x, w1, b1, gamma1, beta1, mean1, var1,
           w2, b2, gamma2, beta2, mean2, var2):
    xn = jnp.transpose(x, (0, 2, 3, 4, 1)).astype(jnp.float32)
    N, D, H, W, C = xn.shape
    band1, t1c = _fold_band(w1, b1, gamma1, beta1, mean1, var1, C)
    band2, t2c = _fold_band(w2, b2, gamma2, beta2, mean2, var2, C)
    NB = 8
    while N % NB:
        NB //= 2
    out = _build_call(N, D, H, W, C, NB)(xn, band1, t1c, band2, t2c)
    return jnp.transpose(out, (0, 4, 1, 2, 3))


# speedup vs baseline: 1.4849x; 1.4849x over previous
import jax
import jax.numpy as jnp
from jax.experimental import pallas as pl
from jax.experimental.pallas import tpu as pltpu

_SLOPE = 0.3
_EPS = 1e-5


def _leaky(v):
    return jnp.where(v >= 0, v, _SLOPE * v)


def _block_kernel(x_ref, w1_ref, t1_ref, w2_ref, t2_ref, o_ref,
                  xw_a, yw_a, xw_b, yw_b):
    NB, D, H, W, C = x_ref.shape
    NQ = W // 2
    KW = 4 * C
    NC = 2 * C
    R = 2 * NQ
    M = D * H * R
    bf16 = jnp.bfloat16

    zd = jnp.zeros((1, H + 2, R, KW), bf16)
    zh = jnp.zeros((D, 1, R, KW), bf16)
    zc = jnp.zeros((D, H, 2, C), bf16)
    for ref in (xw_a, yw_a, xw_b, yw_b):
        ref[0:1, :, :, :] = zd
        ref[D + 1:D + 2, :, :, :] = zd
        ref[1:1 + D, 0:1, :, :] = zh
        ref[1:1 + D, H + 1:H + 2, :, :] = zh
        ref[1:1 + D, 1:1 + H, 0:2, 0:C] = zc
        ref[1:1 + D, 1:1 + H, R - 2:R, KW - C:KW] = zc

    def conv(src_ref, w_ref):
        acc = jnp.zeros((M, NC), jnp.float32)
        for t in range(9):
            kd, kh = t // 3, t % 3
            lhs = src_ref[kd:kd + D, kh:kh + H, :, :].reshape(M, KW)
            acc = acc + jnp.dot(lhs, w_ref[t],
                                preferred_element_type=jnp.float32)
        return acc

    def build_windows(i, xw_ref):
        xv = x_ref[i:i + 2].astype(bf16)
        for q in range(1, NQ - 1):
            win = xv[:, :, :, 2 * q - 1:2 * q + 3, :].reshape(2, D, H, KW)
            xw_ref[1:1 + D, 1:1 + H, 2 * q:2 * q + 2, :] = (
                jnp.transpose(win, (1, 2, 0, 3)))
        w0 = xv[:, :, :, 0:3, :].reshape(2, D, H, 3 * C)
        xw_ref[1:1 + D, 1:1 + H, 0:2, C:KW] = jnp.transpose(w0, (1, 2, 0, 3))
        wl = xv[:, :, :, W - 3:W, :].reshape(2, D, H, 3 * C)
        xw_ref[1:1 + D, 1:1 + H, R - 2:R, 0:3 * C] = jnp.transpose(wl, (1, 2, 0, 3))

    def conv1_scatter(xw_ref, yw_ref):
        y = _leaky(conv(xw_ref, w1_ref) + t1_ref[...])
        yb = y.astype(bf16).reshape(D, H, R, NC)
        yw_ref[1:1 + D, 1:1 + H, :, C:3 * C] = yb
        yc = yb.reshape(D, H, R, 2, C)
        yw_ref[1:1 + D, 1:1 + H, 0:R - 2, 3 * C:KW] = yc[:, :, 2:R, 0, :]
        yw_ref[1:1 + D, 1:1 + H, 2:R, 0:C] = yc[:, :, 0:R - 2, 1, :]

    def conv2_out(i, yw_ref):
        z = conv(yw_ref, w2_ref) + t2_ref[...]
        zs = z.reshape(D, H, NQ, 2, 2, C)
        for n2 in range(2):
            zn = zs[:, :, :, n2, :, :].reshape(D, H, W, C)
            o_ref[i + n2] = _leaky(zn + x_ref[i + n2])

    build_windows(0, xw_a)
    build_windows(2, xw_b)
    conv1_scatter(xw_a, yw_a)
    conv1_scatter(xw_b, yw_b)
    conv2_out(0, yw_a)
    conv2_out(2, yw_b)


def _build_call(N, D, H, W, C, NB):
    NQ = W // 2
    KW, NC = 4 * C, 2 * C
    vol = pl.BlockSpec((NB, D, H, W, C), lambda n: (n, 0, 0, 0, 0))
    wspec = pl.BlockSpec((9, KW, NC), lambda n: (0, 0, 0))
    tspec = pl.BlockSpec((1, NC), lambda n: (0, 0))
    scratch = pltpu.VMEM((D + 2, H + 2, 2 * NQ, KW), jnp.bfloat16)
    return pl.pallas_call(
        _block_kernel,
        out_shape=jax.ShapeDtypeStruct((N, D, H, W, C), jnp.float32),
        grid=(N // NB,),
        in_specs=[vol, wspec, tspec, wspec, tspec],
        out_specs=vol,
        scratch_shapes=[scratch, scratch, scratch, scratch],
        compiler_params=pltpu.CompilerParams(
            dimension_semantics=("parallel",),
            vmem_limit_bytes=52 * 1024 * 1024,
        ),
    )


def _fold_band(w, conv_b, gamma, beta, mean, var, C):
    s = gamma * jax.lax.rsqrt(var + _EPS)
    t = conv_b * s + beta - mean * s
    wt = jnp.transpose(w * s[:, None, None, None, None],
                       (2, 3, 4, 1, 0))
    band = jnp.zeros((3, 3, 4, C, 2, C), jnp.float32)
    for wq in range(2):
        for kw in range(3):
            band = band.at[:, :, wq + kw, :, wq, :].set(wt[:, :, kw])
    band = band.reshape(9, 4 * C, 2 * C).astype(jnp.bfloat16)
    tcol = jnp.concatenate([t, t]).reshape(1, 2 * C).astype(jnp.float32)
    return band, tcol


def kernel(x, w1, b1, gamma1, beta1, mean1, var1,
           w2, b2, gamma2, beta2, mean2, var2):
    xn = jnp.transpose(x, (0, 2, 3, 4, 1)).astype(jnp.float32)
    N, D, H, W, C = xn.shape
    band1, t1c = _fold_band(w1, b1, gamma1, beta1, mean1, var1, C)
    band2, t2c = _fold_band(w2, b2, gamma2, beta2, mean2, var2, C)
    out = _build_call(N, D, H, W, C, 4)(xn, band1, t1c, band2, t2c)
    return jnp.transpose(out, (0, 4, 1, 2, 3))


# speedup vs baseline: 1.6214x; 1.0919x over previous
import jax
import jax.numpy as jnp
from jax.experimental import pallas as pl
from jax.experimental.pallas import tpu as pltpu

_SLOPE = 0.3
_EPS = 1e-5


def _leaky(v):
    return jnp.where(v >= 0, v, _SLOPE * v)


def _block_kernel(x_ref, w1_ref, t1_ref, w2_ref, t2_ref, o_ref, *scratches):
    NB, D, H, W, C = x_ref.shape
    NQ = W // 2
    KW = 4 * C
    NC = 2 * C
    R = 2 * NQ
    M = D * H * R
    bf16 = jnp.bfloat16

    zd = jnp.zeros((1, H + 2, R, KW), bf16)
    zh = jnp.zeros((D, 1, R, KW), bf16)
    zc = jnp.zeros((D, H, 2, C), bf16)
    for ref in scratches:
        ref[0:1, :, :, :] = zd
        ref[D + 1:D + 2, :, :, :] = zd
        ref[1:1 + D, 0:1, :, :] = zh
        ref[1:1 + D, H + 1:H + 2, :, :] = zh
        ref[1:1 + D, 1:1 + H, 0:2, 0:C] = zc
        ref[1:1 + D, 1:1 + H, R - 2:R, KW - C:KW] = zc

    def conv(src_ref, w_ref):
        acc = jnp.zeros((M, NC), jnp.float32)
        for t in range(9):
            kd, kh = t // 3, t % 3
            lhs = src_ref[kd:kd + D, kh:kh + H, :, :].reshape(M, KW)
            acc = acc + jnp.dot(lhs, w_ref[t],
                                preferred_element_type=jnp.float32)
        return acc

    def build_windows(i, xw_ref):
        xv = x_ref[i:i + 2].astype(bf16)
        for q in range(1, NQ - 1):
            win = xv[:, :, :, 2 * q - 1:2 * q + 3, :].reshape(2, D, H, KW)
            xw_ref[1:1 + D, 1:1 + H, 2 * q:2 * q + 2, :] = (
                jnp.transpose(win, (1, 2, 0, 3)))
        w0 = xv[:, :, :, 0:3, :].reshape(2, D, H, 3 * C)
        xw_ref[1:1 + D, 1:1 + H, 0:2, C:KW] = jnp.transpose(w0, (1, 2, 0, 3))
        wl = xv[:, :, :, W - 3:W, :].reshape(2, D, H, 3 * C)
        xw_ref[1:1 + D, 1:1 + H, R - 2:R, 0:3 * C] = jnp.transpose(wl, (1, 2, 0, 3))

    def conv1_scatter(xw_ref, yw_ref):
        y = _leaky(conv(xw_ref, w1_ref) + t1_ref[...])
        yb = y.astype(bf16).reshape(D, H, R, NC)
        yw_ref[1:1 + D, 1:1 + H, :, C:3 * C] = yb
        yc = yb.reshape(D, H, R, 2, C)
        yw_ref[1:1 + D, 1:1 + H, 0:R - 2, 3 * C:KW] = yc[:, :, 2:R, 0, :]
        yw_ref[1:1 + D, 1:1 + H, 2:R, 0:C] = yc[:, :, 0:R - 2, 1, :]

    def conv2_out(i, yw_ref):
        z = conv(yw_ref, w2_ref) + t2_ref[...]
        zs = z.reshape(D, H, NQ, 2, 2, C)
        for n2 in range(2):
            zn = zs[:, :, :, n2, :, :].reshape(D, H, W, C)
            o_ref[i + n2] = _leaky(zn + x_ref[i + n2])

    npairs = NB // 2
    xws = scratches[0::2]
    yws = scratches[1::2]
    for p in range(npairs):
        build_windows(2 * p, xws[p])
    for p in range(npairs):
        conv1_scatter(xws[p], yws[p])
    for p in range(npairs):
        conv2_out(2 * p, yws[p])


def _build_call(N, D, H, W, C, NB):
    NQ = W // 2
    KW, NC = 4 * C, 2 * C
    vol = pl.BlockSpec((NB, D, H, W, C), lambda n: (n, 0, 0, 0, 0))
    wspec = pl.BlockSpec((9, KW, NC), lambda n: (0, 0, 0))
    tspec = pl.BlockSpec((1, NC), lambda n: (0, 0))
    scratch = pltpu.VMEM((D + 2, H + 2, 2 * NQ, KW), jnp.bfloat16)
    return pl.pallas_call(
        _block_kernel,
        out_shape=jax.ShapeDtypeStruct((N, D, H, W, C), jnp.float32),
        grid=(N // NB,),
        in_specs=[vol, wspec, tspec, wspec, tspec],
        out_specs=vol,
        scratch_shapes=[scratch] * NB,
        compiler_params=pltpu.CompilerParams(
            dimension_semantics=("parallel",),
            vmem_limit_bytes=52 * 1024 * 1024,
        ),
    )


def _fold_band(w, conv_b, gamma, beta, mean, var, C):
    s = gamma * jax.lax.rsqrt(var + _EPS)
    t = conv_b * s + beta - mean * s
    wt = jnp.transpose(w * s[:, None, None, None, None],
                       (2, 3, 4, 1, 0))
    band = jnp.zeros((3, 3, 4, C, 2, C), jnp.float32)
    for wq in range(2):
        for kw in range(3):
            band = band.at[:, :, wq + kw, :, wq, :].set(wt[:, :, kw])
    band = band.reshape(9, 4 * C, 2 * C).astype(jnp.bfloat16)
    tcol = jnp.concatenate([t, t]).reshape(1, 2 * C).astype(jnp.float32)
    return band, tcol


def kernel(x, w1, b1, gamma1, beta1, mean1, var1,
           w2, b2, gamma2, beta2, mean2, var2):
    xn = jnp.transpose(x, (0, 2, 3, 4, 1)).astype(jnp.float32)
    N, D, H, W, C = xn.shape
    band1, t1c = _fold_band(w1, b1, gamma1, beta1, mean1, var1, C)
    band2, t2c = _fold_band(w2, b2, gamma2, beta2, mean2, var2, C)
    NB = 8 if N % 8 == 0 else (4 if N % 4 == 0 else 2)
    out = _build_call(N, D, H, W, C, NB)(xn, band1, t1c, band2, t2c)
    return jnp.transpose(out, (0, 4, 1, 2, 3))


# speedup vs baseline: 1.6496x; 1.0174x over previous
import jax
import jax.numpy as jnp
from jax.experimental import pallas as pl
from jax.experimental.pallas import tpu as pltpu

_SLOPE = 0.3
_EPS = 1e-5


def _leaky(v):
    return jnp.where(v >= 0, v, _SLOPE * v)


def _block_kernel(x_ref, w1_ref, t1_ref, w2_ref, t2_ref, o_ref, *scratches):
    NB, D, H, W, C = x_ref.shape
    NQ = W // 2
    KW = 4 * C
    NC = 2 * C
    R = 2 * NQ
    M = D * H * R
    bf16 = jnp.bfloat16

    @pl.when(pl.program_id(1) == 0)
    def _zero_halos():
        zd = jnp.zeros((1, H + 2, R, KW), bf16)
        zh = jnp.zeros((D, 1, R, KW), bf16)
        zc = jnp.zeros((D, H, 2, C), bf16)
        for ref in scratches:
            ref[0:1, :, :, :] = zd
            ref[D + 1:D + 2, :, :, :] = zd
            ref[1:1 + D, 0:1, :, :] = zh
            ref[1:1 + D, H + 1:H + 2, :, :] = zh
            ref[1:1 + D, 1:1 + H, 0:2, 0:C] = zc
            ref[1:1 + D, 1:1 + H, R - 2:R, KW - C:KW] = zc

    def conv(src_ref, w_ref):
        acc = jnp.zeros((M, NC), jnp.float32)
        for t in range(9):
            kd, kh = t // 3, t % 3
            lhs = src_ref[kd:kd + D, kh:kh + H, :, :].reshape(M, KW)
            acc = acc + jnp.dot(lhs, w_ref[t],
                                preferred_element_type=jnp.float32)
        return acc

    def build_windows(i, xw_ref):
        xv = x_ref[i:i + 2].astype(bf16)
        for q in range(1, NQ - 1):
            win = xv[:, :, :, 2 * q - 1:2 * q + 3, :].reshape(2, D, H, KW)
            xw_ref[1:1 + D, 1:1 + H, 2 * q:2 * q + 2, :] = (
                jnp.transpose(win, (1, 2, 0, 3)))
        w0 = xv[:, :, :, 0:3, :].reshape(2, D, H, 3 * C)
        xw_ref[1:1 + D, 1:1 + H, 0:2, C:KW] = jnp.transpose(w0, (1, 2, 0, 3))
        wl = xv[:, :, :, W - 3:W, :].reshape(2, D, H, 3 * C)
        xw_ref[1:1 + D, 1:1 + H, R - 2:R, 0:3 * C] = jnp.transpose(wl, (1, 2, 0, 3))

    def conv1_scatter(xw_ref, yw_ref):
        y = _leaky(conv(xw_ref, w1_ref) + t1_ref[...])
        yb = y.astype(bf16).reshape(D, H, R, NC)
        yw_ref[1:1 + D, 1:1 + H, :, C:3 * C] = yb
        yc = yb.reshape(D, H, R, 2, C)
        yw_ref[1:1 + D, 1:1 + H, 0:R - 2, 3 * C:KW] = yc[:, :, 2:R, 0, :]
        yw_ref[1:1 + D, 1:1 + H, 2:R, 0:C] = yc[:, :, 0:R - 2, 1, :]

    def conv2_out(i, yw_ref):
        z = conv(yw_ref, w2_ref) + t2_ref[...]
        zs = z.reshape(D, H, NQ, 2, 2, C)
        for n2 in range(2):
            zn = zs[:, :, :, n2, :, :].reshape(D, H, W, C)
            o_ref[i + n2] = _leaky(zn + x_ref[i + n2])

    npairs = NB // 2
    xws = scratches[0::2]
    yws = scratches[1::2]
    for p in range(npairs):
        build_windows(2 * p, xws[p])
    for p in range(npairs):
        conv1_scatter(xws[p], yws[p])
    for p in range(npairs):
        conv2_out(2 * p, yws[p])


def _build_call(N, D, H, W, C, NB):
    NQ = W // 2
    KW, NC = 4 * C, 2 * C
    steps = N // NB
    half = max(steps // 2, 1)
    ncore = steps // half
    vol = pl.BlockSpec((NB, D, H, W, C), lambda i, j: (i * half + j, 0, 0, 0, 0))
    wspec = pl.BlockSpec((9, KW, NC), lambda i, j: (0, 0, 0))
    tspec = pl.BlockSpec((1, NC), lambda i, j: (0, 0))
    scratch = pltpu.VMEM((D + 2, H + 2, 2 * NQ, KW), jnp.bfloat16)
    return pl.pallas_call(
        _block_kernel,
        out_shape=jax.ShapeDtypeStruct((N, D, H, W, C), jnp.float32),
        grid=(ncore, half),
        in_specs=[vol, wspec, tspec, wspec, tspec],
        out_specs=vol,
        scratch_shapes=[scratch] * NB,
        compiler_params=pltpu.CompilerParams(
            dimension_semantics=("parallel", "arbitrary"),
            vmem_limit_bytes=52 * 1024 * 1024,
        ),
    )


def _fold_band(w, conv_b, gamma, beta, mean, var, C):
    s = gamma * jax.lax.rsqrt(var + _EPS)
    t = conv_b * s + beta - mean * s
    wt = jnp.transpose(w * s[:, None, None, None, None],
                       (2, 3, 4, 1, 0))
    band = jnp.zeros((3, 3, 4, C, 2, C), jnp.float32)
    for wq in range(2):
        for kw in range(3):
            band = band.at[:, :, wq + kw, :, wq, :].set(wt[:, :, kw])
    band = band.reshape(9, 4 * C, 2 * C).astype(jnp.bfloat16)
    tcol = jnp.concatenate([t, t]).reshape(1, 2 * C).astype(jnp.float32)
    return band, tcol


def kernel(x, w1, b1, gamma1, beta1, mean1, var1,
           w2, b2, gamma2, beta2, mean2, var2):
    xn = jnp.transpose(x, (0, 2, 3, 4, 1)).astype(jnp.float32)
    N, D, H, W, C = xn.shape
    band1, t1c = _fold_band(w1, b1, gamma1, beta1, mean1, var1, C)
    band2, t2c = _fold_band(w2, b2, gamma2, beta2, mean2, var2, C)
    NB = 8 if N % 8 == 0 else (4 if N % 4 == 0 else 2)
    out = _build_call(N, D, H, W, C, NB)(xn, band1, t1c, band2, t2c)
    return jnp.transpose(out, (0, 4, 1, 2, 3))
